# baseline (speedup 1.0000x reference)
import jax
import jax.numpy as jnp
from jax import lax
from jax.experimental import pallas as pl
from jax.experimental.pallas import tpu as pltpu
from jax.experimental.pallas import tpu_sc as plsc

N = 10000
NP = 10240
E = 320000
D = 128
CH = 80
KB = 16
NSL = 4
NC, NS = 2, 16
NW = NC * NS
CPW = 128
NB = CPW // KB
EP = CPW * NW * CH
RPT = NP // NS


def _make_sc_agg(gather: bool):
  mesh = plsc.VectorSubcoreMesh(core_axis_name="c", subcore_axis_name="s")

  def body(*refs):
    if gather:
      (table, src, dst, zseed, acc_out, sblk0, sblk1) = refs[:7]
      rest = refs[7:]
    else:
      (dst, zseed, oseed, acc_out) = refs[:4]
      rest = refs[4:]
      sblk0 = sblk1 = src = table = None
    didx = rest[0:NSL]
    rows = rest[NSL:2 * NSL]
    acc = rest[2 * NSL]
    dsem = rest[2 * NSL + 1:3 * NSL + 1]
    gsem = rest[3 * NSL + 1:4 * NSL + 1]
    ssem = rest[4 * NSL + 1:5 * NSL + 1]
    sbsem = rest[5 * NSL + 1:5 * NSL + 3]
    rows0 = rows[0]
    c = lax.axis_index("c")
    s = lax.axis_index("s")
    wid = s * NC + c

    fills = [pltpu.async_copy(zseed, rows0.at[pl.ds(k * 8, 8)], dsem[0])
             for k in range(CH // 8)]
    for f in fills:
      f.wait()
    zcs = [pltpu.async_copy(rows0, acc.at[pl.ds(s * RPT + k * CH, CH)],
                            dsem[1])
           for k in range(RPT // CH)]
    for f in zcs:
      f.wait()
    if not gather:
      fills = [pltpu.async_copy(oseed, r.at[pl.ds(k * 8, 8)], dsem[0])
               for k in range(CH // 8) for r in rows]
      for f in fills:
        f.wait()
    plsc.subcore_barrier()

    def run_block(jblk, sblk, dds, gaths, scats):
      cbase = pl.multiple_of(jblk * (KB * CH), KB * CH)
      for q in range(KB + 2):
        b = q % NSL
        if q < KB:
          if scats[b] is not None:
            scats[b].wait()
          dds[b] = pltpu.async_copy(dst.at[pl.ds(cbase + q * CH, CH)],
                                    didx[b], dsem[b])
          if gather:
            gaths[b] = pltpu.async_copy(
                table.at[sblk.at[pl.ds(q * CH, CH)]], rows[b], gsem[b])
        if q >= 2:
          bb = (q - 2) % NSL
          if gather:
            gaths[bb].wait()
          dds[bb].wait()
          scats[bb] = pltpu.async_copy(rows[bb], acc.at[didx[bb]],
                                       ssem[bb], add=True)

    def load_sblk(jblk, sblk, sem):
      jc = jnp.minimum(jblk, wid * NB + (NB - 1))
      base = pl.multiple_of(jc * (KB * CH), KB * CH)
      return pltpu.async_copy(src.at[pl.ds(base, KB * CH)], sblk, sem)

    if gather:
      l0 = load_sblk(wid * NB, sblk0, sbsem[0])

      def superblock(t, carry):
        j0 = wid * NB + 2 * t
        l1 = load_sblk(j0 + 1, sblk1, sbsem[1])
        l0.wait()
        dds = [None] * NSL
        gaths = [None] * NSL
        scats = [None] * NSL
        run_block(j0, sblk0, dds, gaths, scats)
        lnext = load_sblk(j0 + 2, sblk0, sbsem[0])
        l1.wait()
        run_block(j0 + 1, sblk1, dds, gaths, scats)
        for b in range(NSL):
          scats[b].wait()
        return carry

      lax.fori_loop(0, NB // 2, superblock, 0)
      pltpu.make_async_copy(src.at[pl.ds(0, KB * CH)], sblk0,
                            sbsem[0]).wait()
    else:
      def block(j, carry):
        dds = [None] * NSL
        gaths = [None] * NSL
        scats = [None] * NSL
        run_block(wid * NB + j, None, dds, gaths, scats)
        for b in range(NSL):
          scats[b].wait()
        return carry

      lax.fori_loop(0, NB, block, 0)
    plsc.subcore_barrier()

    nwb = RPT // CH
    ins = [None] * NSL
    outs = [None] * NSL
    for k in range(nwb + 2):
      b = k % NSL
      if k < nwb:
        if outs[b] is not None:
          outs[b].wait()
        ins[b] = pltpu.async_copy(acc.at[pl.ds(s * RPT + k * CH, CH)],
                                  rows[b], gsem[b])
      if k >= 2:
        bb = (k - 2) % NSL
        ins[bb].wait()
        off = s * RPT + (k - 2) * CH
        outs[bb] = pltpu.async_copy(rows[bb], acc_out.at[c, pl.ds(off, CH)],
                                    ssem[bb])
    for b in range(NSL):
      if outs[b] is not None:
        outs[b].wait()

  scratch = (
      [pltpu.VMEM((CH,), jnp.int32) for _ in range(NSL)]
      + [pltpu.VMEM((CH, D), jnp.float32) for _ in range(NSL)]
      + [pltpu.VMEM_SHARED((NP, D), jnp.float32)]
      + [pltpu.SemaphoreType.DMA for _ in range(3 * NSL + 2)]
  )
  if gather:
    scratch.insert(0, pltpu.VMEM((KB * CH,), jnp.int32))
    scratch.insert(0, pltpu.VMEM((KB * CH,), jnp.int32))

  return pl.kernel(body,
                   out_type=jax.ShapeDtypeStruct((NC, NP, D), jnp.float32),
                   mesh=mesh, scratch_types=scratch)


_sc_agg = _make_sc_agg(True)
_sc_cnt = _make_sc_agg(False)


def _make_tc_layer(relu: bool):
  R = 1280

  def body(a0, a1, c0, c1, h, wl, wr, b, o):
    cnt = c0[:, :1] + c1[:, :1]
    inv = 1.0 / jnp.maximum(cnt, 1.0)
    agg = (a0[...] + a1[...]) * inv
    y = jnp.dot(agg, wl[...], preferred_element_type=jnp.float32)
    y = y + jnp.dot(h[...], wr[...], preferred_element_type=jnp.float32)
    y = y + b[...]
    if relu:
      y = jnp.maximum(y, 0.0)
    o[...] = y

  row = lambda i: (i, 0)
  zero = lambda i: (0, 0)
  return pl.pallas_call(
      body,
      grid=(NP // R,),
      in_specs=[
          pl.BlockSpec((R, D), row),
          pl.BlockSpec((R, D), row),
          pl.BlockSpec((R, D), row),
          pl.BlockSpec((R, D), row),
          pl.BlockSpec((R, D), row),
          pl.BlockSpec((D, D), zero),
          pl.BlockSpec((D, D), zero),
          pl.BlockSpec((1, D), zero),
      ],
      out_specs=pl.BlockSpec((R, D), row),
      out_shape=jax.ShapeDtypeStruct((NP, D), jnp.float32),
  )


_tc_relu = _make_tc_layer(True)
_tc_lin = _make_tc_layer(False)


@jax.jit
def kernel(x, edge_index, W1l, b1l, W1r, W2l, b2l, W2r, W3l, b3l, W3r):
  epw = E // NW
  ppw = EP // NW - epw
  pad_src = jnp.broadcast_to(jnp.arange(ppw, dtype=jnp.int32), (NW, ppw))
  pad_dst = jnp.broadcast_to(N + jnp.arange(ppw, dtype=jnp.int32) % (NP - N),
                             (NW, ppw))
  src = jnp.concatenate([edge_index[0].reshape(NW, epw), pad_src],
                        axis=1).reshape(-1)
  dst = jnp.concatenate([edge_index[1].reshape(NW, epw), pad_dst],
                        axis=1).reshape(-1)
  xp = jnp.pad(x, ((0, NP - N), (0, 0)))
  zseed = jnp.zeros((8, D), jnp.float32)
  oseed = jnp.ones((8, D), jnp.float32)

  cntp = _sc_cnt(dst, zseed, oseed)
  acc1 = _sc_agg(xp, src, dst, zseed)
  h1 = _tc_relu(acc1[0], acc1[1], cntp[0], cntp[1], xp,
                W1l, W1r, b1l.reshape(1, D))
  acc2 = _sc_agg(h1, src, dst, zseed)
  h2 = _tc_relu(acc2[0], acc2[1], cntp[0], cntp[1], h1,
                W2l, W2r, b2l.reshape(1, D))
  acc3 = _sc_agg(h2, src, dst, zseed)
  out = _tc_lin(acc3[0], acc3[1], cntp[0], cntp[1], h2,
                W3l, W3r, b3l.reshape(1, D))
  return out[:N]


# speedup vs baseline: 1.0197x; 1.0197x over previous
import jax
import jax.numpy as jnp
from jax import lax
from jax.experimental import pallas as pl
from jax.experimental.pallas import tpu as pltpu
from jax.experimental.pallas import tpu_sc as plsc

N = 10000
NP = 10240
E = 320000
D = 128
CH = 64
KB = 16
NSL = 4
NC, NS = 2, 16
NW = NC * NS
CPW = 160
NB = CPW // KB
EP = CPW * NW * CH
RPT = NP // NS


def _make_sc_agg(gather: bool):
  mesh = plsc.VectorSubcoreMesh(core_axis_name="c", subcore_axis_name="s")

  def body(*refs):
    if gather:
      (table, src, dst, zseed, acc_out, sblk0, sblk1) = refs[:7]
      rest = refs[7:]
    else:
      (dst, zseed, oseed, acc_out) = refs[:4]
      rest = refs[4:]
      sblk0 = sblk1 = src = table = None
    didx = rest[0:NSL]
    rows = rest[NSL:2 * NSL]
    acc = rest[2 * NSL]
    dsem = rest[2 * NSL + 1:3 * NSL + 1]
    gsem = rest[3 * NSL + 1:4 * NSL + 1]
    ssem = rest[4 * NSL + 1:5 * NSL + 1]
    sbsem = rest[5 * NSL + 1:5 * NSL + 3]
    rows0 = rows[0]
    c = lax.axis_index("c")
    s = lax.axis_index("s")
    wid = s * NC + c

    fills = [pltpu.async_copy(zseed, rows0.at[pl.ds(k * 8, 8)], dsem[0])
             for k in range(CH // 8)]
    for f in fills:
      f.wait()
    zcs = [pltpu.async_copy(rows0, acc.at[pl.ds(s * RPT + k * CH, CH)],
                            dsem[1])
           for k in range(RPT // CH)]
    for f in zcs:
      f.wait()
    if not gather:
      fills = [pltpu.async_copy(oseed, r.at[pl.ds(k * 8, 8)], dsem[0])
               for k in range(CH // 8) for r in rows]
      for f in fills:
        f.wait()
    plsc.subcore_barrier()

    def run_block(jblk, sblk, dds, gaths, scats):
      cbase = pl.multiple_of(jblk * (KB * CH), KB * CH)
      for q in range(KB + 2):
        b = q % NSL
        if q < KB:
          if scats[b] is not None:
            scats[b].wait()
          dds[b] = pltpu.async_copy(dst.at[pl.ds(cbase + q * CH, CH)],
                                    didx[b], dsem[b])
          if gather:
            gaths[b] = pltpu.async_copy(
                table.at[sblk.at[pl.ds(q * CH, CH)]], rows[b], gsem[b])
        if q >= 2:
          bb = (q - 2) % NSL
          if gather:
            gaths[bb].wait()
          dds[bb].wait()
          scats[bb] = pltpu.async_copy(rows[bb], acc.at[didx[bb]],
                                       ssem[bb], add=True)

    def load_sblk(jblk, sblk, sem):
      jc = jnp.minimum(jblk, wid * NB + (NB - 1))
      base = pl.multiple_of(jc * (KB * CH), KB * CH)
      return pltpu.async_copy(src.at[pl.ds(base, KB * CH)], sblk, sem)

    if gather:
      l0 = load_sblk(wid * NB, sblk0, sbsem[0])

      def superblock(t, carry):
        j0 = wid * NB + 2 * t
        l1 = load_sblk(j0 + 1, sblk1, sbsem[1])
        l0.wait()
        dds = [None] * NSL
        gaths = [None] * NSL
        scats = [None] * NSL
        run_block(j0, sblk0, dds, gaths, scats)
        lnext = load_sblk(j0 + 2, sblk0, sbsem[0])
        l1.wait()
        run_block(j0 + 1, sblk1, dds, gaths, scats)
        for b in range(NSL):
          scats[b].wait()
        return carry

      lax.fori_loop(0, NB // 2, superblock, 0)
      pltpu.make_async_copy(src.at[pl.ds(0, KB * CH)], sblk0,
                            sbsem[0]).wait()
    else:
      def block(j, carry):
        dds = [None] * NSL
        gaths = [None] * NSL
        scats = [None] * NSL
        run_block(wid * NB + j, None, dds, gaths, scats)
        for b in range(NSL):
          scats[b].wait()
        return carry

      lax.fori_loop(0, NB, block, 0)
    plsc.subcore_barrier()

    nwb = RPT // CH
    ins = [None] * NSL
    outs = [None] * NSL
    for k in range(nwb + 2):
      b = k % NSL
      if k < nwb:
        if outs[b] is not None:
          outs[b].wait()
        ins[b] = pltpu.async_copy(acc.at[pl.ds(s * RPT + k * CH, CH)],
                                  rows[b], gsem[b])
      if k >= 2:
        bb = (k - 2) % NSL
        ins[bb].wait()
        off = s * RPT + (k - 2) * CH
        outs[bb] = pltpu.async_copy(rows[bb], acc_out.at[c, pl.ds(off, CH)],
                                    ssem[bb])
    for b in range(NSL):
      if outs[b] is not None:
        outs[b].wait()

  scratch = (
      [pltpu.VMEM((CH,), jnp.int32) for _ in range(NSL)]
      + [pltpu.VMEM((CH, D), jnp.float32) for _ in range(NSL)]
      + [pltpu.VMEM_SHARED((NP, D), jnp.float32)]
      + [pltpu.SemaphoreType.DMA for _ in range(3 * NSL + 2)]
  )
  if gather:
    scratch.insert(0, pltpu.VMEM((KB * CH,), jnp.int32))
    scratch.insert(0, pltpu.VMEM((KB * CH,), jnp.int32))

  return pl.kernel(body,
                   out_type=jax.ShapeDtypeStruct((NC, NP, D), jnp.float32),
                   mesh=mesh, scratch_types=scratch)


_sc_agg = _make_sc_agg(True)
_sc_cnt = _make_sc_agg(False)


def _make_tc_layer(relu: bool):
  R = 1280

  def body(a0, a1, c0, c1, h, wl, wr, b, o):
    cnt = c0[:, :1] + c1[:, :1]
    inv = 1.0 / jnp.maximum(cnt, 1.0)
    agg = (a0[...] + a1[...]) * inv
    y = jnp.dot(agg, wl[...], preferred_element_type=jnp.float32)
    y = y + jnp.dot(h[...], wr[...], preferred_element_type=jnp.float32)
    y = y + b[...]
    if relu:
      y = jnp.maximum(y, 0.0)
    o[...] = y

  row = lambda i: (i, 0)
  zero = lambda i: (0, 0)
  return pl.pallas_call(
      body,
      grid=(NP // R,),
      in_specs=[
          pl.BlockSpec((R, D), row),
          pl.BlockSpec((R, D), row),
          pl.BlockSpec((R, D), row),
          pl.BlockSpec((R, D), row),
          pl.BlockSpec((R, D), row),
          pl.BlockSpec((D, D), zero),
          pl.BlockSpec((D, D), zero),
          pl.BlockSpec((1, D), zero),
      ],
      out_specs=pl.BlockSpec((R, D), row),
      out_shape=jax.ShapeDtypeStruct((NP, D), jnp.float32),
  )


_tc_relu = _make_tc_layer(True)
_tc_lin = _make_tc_layer(False)


@jax.jit
def kernel(x, edge_index, W1l, b1l, W1r, W2l, b2l, W2r, W3l, b3l, W3r):
  epw = E // NW
  ppw = EP // NW - epw
  pad_src = jnp.broadcast_to(jnp.arange(ppw, dtype=jnp.int32), (NW, ppw))
  pad_dst = jnp.broadcast_to(N + jnp.arange(ppw, dtype=jnp.int32) % (NP - N),
                             (NW, ppw))
  src = jnp.concatenate([edge_index[0].reshape(NW, epw), pad_src],
                        axis=1).reshape(-1)
  dst = jnp.concatenate([edge_index[1].reshape(NW, epw), pad_dst],
                        axis=1).reshape(-1)
  xp = jnp.pad(x, ((0, NP - N), (0, 0)))
  zseed = jnp.zeros((8, D), jnp.float32)
  oseed = jnp.ones((8, D), jnp.float32)

  cntp = _sc_cnt(dst, zseed, oseed)
  acc1 = _sc_agg(xp, src, dst, zseed)
  h1 = _tc_relu(acc1[0], acc1[1], cntp[0], cntp[1], xp,
                W1l, W1r, b1l.reshape(1, D))
  acc2 = _sc_agg(h1, src, dst, zseed)
  h2 = _tc_relu(acc2[0], acc2[1], cntp[0], cntp[1], h1,
                W2l, W2r, b2l.reshape(1, D))
  acc3 = _sc_agg(h2, src, dst, zseed)
  out = _tc_lin(acc3[0], acc3[1], cntp[0], cntp[1], h2,
                W3l, W3r, b3l.reshape(1, D))
  return out[:N]
